# baseline (speedup 1.0000x reference)
import functools

import jax
import jax.numpy as jnp
from jax import lax
from jax.experimental import pallas as pl
from jax.experimental.pallas import tpu as pltpu
from jax.experimental.pallas import tpu_sc as plsc

NUM_GSPS = 360
B = 1024
N_PV = 512
F = 8
EMBED_DIM = 64
OUT_D = 2 * F + F + 2 + EMBED_DIM
VOCAB = 100000
VSLICE = 99000

NC = 2
NS = 16
NW = NC * NS
ROWS = B * N_PV
CH = 4096
NCH = ROWS // CH
BPW = B // NW
U = 8
YR = 8
NYT = BPW * F // YR


def _sc_body(y_hbm, x_hbm, idx_hbm, t_hbm, az_hbm, el_hbm, table_hbm,
             out_hbm, rowbuf, idxbuf, gatbuf, planebuf, ystage, scalbuf,
             yidx, xidx, isem, osem, ysem, ssem, bsem):
    wid = lax.axis_index("s") * NC + lax.axis_index("c")
    b0 = wid * BPW
    lane = lax.iota(jnp.int32, 16)

    for g in range(BPW * F // 16):
        r = b0 * F + g * 16 + lane
        bvec = lax.shift_right_logical(r, 3)
        fvec = jnp.bitwise_and(r, 7)
        rowv = jnp.full((16,), 2 * g, jnp.int32) + lax.shift_right_logical(
            lane, 3)
        colv = jnp.bitwise_and(lane, 7)
        plsc.store_scatter(yidx, [rowv, colv], fvec * B + bvec)
        plsc.store_scatter(xidx, [rowv, colv], (fvec + F) * B + bvec)

    def yx_planes(src_hbm, idxref):
        def chunk(t, p):
            @pl.when(t >= 2)
            def _():
                pltpu.make_async_copy(
                    ystage.at[p], out_hbm.at[idxref.at[0]], ysem.at[p]).wait()

            pltpu.async_copy(src_hbm.at[pl.ds(b0 * F + t * YR, YR)],
                             ystage.at[p], ssem.at[p])
            pltpu.make_async_copy(
                src_hbm.at[pl.ds(b0 * F + t * YR, YR)], ystage.at[p],
                ssem.at[p]).wait()
            pltpu.async_copy(ystage.at[p], out_hbm.at[idxref.at[t]],
                             ysem.at[p])

        def two(h, carry):
            chunk(2 * h, 0)
            chunk(2 * h + 1, 1)
            return carry

        lax.fori_loop(0, NYT // 2, two, 0, unroll=False)
        for p in range(2):
            pltpu.make_async_copy(
                ystage.at[p], out_hbm.at[idxref.at[0]], ysem.at[p]).wait()

    yx_planes(y_hbm, yidx)
    yx_planes(x_hbm, xidx)

    pltpu.sync_copy(t_hbm.at[:, pl.ds(b0, BPW)], scalbuf.at[pl.ds(0, F)])
    pltpu.sync_copy(az_hbm.at[pl.ds(b0, BPW)], scalbuf.at[F])
    pltpu.sync_copy(el_hbm.at[pl.ds(b0, BPW)], scalbuf.at[F + 1])

    BR = 4
    NBLK = BPW // BR

    def bblock(i, p):
        pi = lax.div(i, NBLK)
        blk = lax.rem(i, NBLK)

        @pl.when(i >= 2)
        def _():
            pltpu.make_async_copy(
                planebuf.at[p], out_hbm.at[pl.ds(0, BR), :],
                bsem.at[p]).wait()

        def fill_row(r, carry):
            v = plsc.load_gather(
                scalbuf, [jnp.full((16,), pi, jnp.int32),
                          jnp.full((16,), blk * BR + r, jnp.int32)])
            for k in range(N_PV // 16):
                planebuf[p, r, pl.ds(k * 16, 16)] = v
            return carry

        lax.fori_loop(0, BR, fill_row, 0, unroll=False)
        pltpu.async_copy(
            planebuf.at[p],
            out_hbm.at[pl.ds((2 * F + pi) * B + b0 + blk * BR, BR), :],
            bsem.at[p])

    def btwo(h, carry):
        bblock(2 * h, 0)
        bblock(2 * h + 1, 1)
        return carry

    lax.fori_loop(0, 10 * NBLK // 2, btwo, 0, unroll=False)

    def fire_idx(c, p):
        pltpu.async_copy(idx_hbm.at[pl.ds(c * CH, CH)], idxbuf.at[p],
                         isem.at[p])

    def wait_idx(c, p):
        pltpu.make_async_copy(idx_hbm.at[pl.ds(c * CH, CH)], idxbuf.at[p],
                              isem.at[p]).wait()

    def fire_out(plane, c, p):
        pltpu.async_copy(gatbuf.at[p],
                         out_hbm.at[pl.ds(plane * B + c * (CH // N_PV),
                                          CH // N_PV), :],
                         osem.at[p])

    def wait_out(p):
        pltpu.make_async_copy(gatbuf.at[p],
                              out_hbm.at[pl.ds(0, CH // N_PV), :],
                              osem.at[p]).wait()

    def gather_chunk(p):
        def body(ko, carry):
            for u in range(U):
                k = ko * U + u
                iv = idxbuf[p, pl.ds(k * 16, 16)]
                vals = plsc.load_gather(rowbuf, [iv])
                row = lax.shift_right_logical(k, 5)
                col = jnp.bitwise_and(k, 31) * 16
                gatbuf[p, row, pl.ds(col, 16)] = vals
            return carry

        lax.fori_loop(0, CH // 16 // U, body, 0, unroll=False)

    def emb_plane(e_row, plane):
        pltpu.sync_copy(table_hbm.at[e_row, pl.ds(NUM_GSPS, VSLICE)],
                        rowbuf.at[pl.ds(0, VSLICE)])
        fire_idx(0, 0)

        def pair(g, carry):
            c0 = 2 * g
            fire_idx(c0 + 1, 1)
            wait_idx(c0, 0)

            @pl.when(c0 >= 2)
            def _():
                wait_out(0)

            gather_chunk(0)
            fire_out(plane, c0, 0)

            @pl.when(c0 + 2 < NCH)
            def _():
                fire_idx(c0 + 2, 0)

            wait_idx(c0 + 1, 1)

            @pl.when(c0 + 1 >= 2)
            def _():
                wait_out(1)

            gather_chunk(1)
            fire_out(plane, c0 + 1, 1)
            return carry

        lax.fori_loop(0, NCH // 2, pair, 0, unroll=False)
        wait_out(0)
        wait_out(1)

    e0 = 2 * wid
    emb_plane(e0, 3 * F + 2 + e0)
    emb_plane(e0 + 1, 3 * F + 2 + e0 + 1)

    for p in range(2):
        pltpu.make_async_copy(planebuf.at[p], out_hbm.at[pl.ds(0, BR), :],
                              bsem.at[p]).wait()


@functools.partial(jax.jit, static_argnames=("interpret",))
def _pv_query(y_t, x_t, idx_flat, t_t, az, el, table_t, interpret=False):
    mesh = plsc.VectorSubcoreMesh(core_axis_name="c", subcore_axis_name="s",
                                  num_cores=NC, num_subcores=NS)
    fn = pl.kernel(
        _sc_body,
        out_type=jax.ShapeDtypeStruct((OUT_D * B, N_PV), jnp.float32),
        mesh=mesh,
        scratch_types=[
            pltpu.VMEM((VSLICE,), jnp.float32),
            pltpu.VMEM((2, CH), jnp.int32),
            pltpu.VMEM((2, CH // N_PV, N_PV), jnp.float32),
            pltpu.VMEM((2, 4, N_PV), jnp.float32),
            pltpu.VMEM((2, YR, N_PV), jnp.float32),
            pltpu.VMEM((F + 2, BPW), jnp.float32),
            pltpu.VMEM((NYT, YR), jnp.int32),
            pltpu.VMEM((NYT, YR), jnp.int32),
            pltpu.SemaphoreType.DMA((2,)),
            pltpu.SemaphoreType.DMA((2,)),
            pltpu.SemaphoreType.DMA((2,)),
            pltpu.SemaphoreType.DMA((2,)),
            pltpu.SemaphoreType.DMA((2,)),
        ],
        compiler_params=pltpu.CompilerParams(use_tc_tiling_on_sc=False,
                                             needs_layout_passes=False),
        interpret=interpret,
    )
    return fn(y_t, x_t, idx_flat, t_t, az, el, table_t)


def kernel(pv_y_osgb_fourier, pv_x_osgb_fourier, pv_system_row_number,
           pv_x_osgb, pv_time_utc_fourier, solar_azimuth, solar_elevation,
           embedding_table):
    del pv_x_osgb
    y_t = jnp.transpose(pv_y_osgb_fourier, (0, 2, 1)).reshape(B * F, N_PV)
    x_t = jnp.transpose(pv_x_osgb_fourier, (0, 2, 1)).reshape(B * F, N_PV)
    idx_flat = pv_system_row_number.astype(jnp.int32).reshape(ROWS)
    t_t = jnp.transpose(pv_time_utc_fourier)
    table_t = jnp.transpose(embedding_table)
    out = _pv_query(y_t, x_t, idx_flat, t_t, solar_azimuth, solar_elevation,
                    table_t)
    return jnp.transpose(out.reshape(OUT_D, B, N_PV), (1, 2, 0))


# speedup vs baseline: 1.4385x; 1.4385x over previous
import functools

import jax
import jax.numpy as jnp
from jax import lax
from jax.experimental import pallas as pl
from jax.experimental.pallas import tpu as pltpu
from jax.experimental.pallas import tpu_sc as plsc

NUM_GSPS = 360
B = 1024
N_PV = 512
F = 8
EMBED_DIM = 64
OUT_D = 2 * F + F + 2 + EMBED_DIM
VOCAB = 100000
VSLICE = 99000

NC = 2
NS = 16
NW = NC * NS
ROWS = B * N_PV
CH = 4096
NCH = ROWS // CH
BPW = B // NW
U = 8
YR = 8
NYT = BPW * F // YR


def _sc_body(y_hbm, x_hbm, idx_hbm, t_hbm, az_hbm, el_hbm, table_hbm,
             out_hbm, rowbuf, idxbuf, gatbuf, planebuf, ystage, scalbuf,
             yidx, xidx, isem, osem, ysem, ssem, bsem):
    wid = lax.axis_index("s") * NC + lax.axis_index("c")
    b0 = wid * BPW
    lane = lax.iota(jnp.int32, 16)

    for g in range(BPW * F // 16):
        r = b0 * F + g * 16 + lane
        bvec = lax.shift_right_logical(r, 3)
        fvec = jnp.bitwise_and(r, 7)
        rowv = jnp.full((16,), 2 * g, jnp.int32) + lax.shift_right_logical(
            lane, 3)
        colv = jnp.bitwise_and(lane, 7)
        plsc.store_scatter(yidx, [rowv, colv], fvec * B + bvec)
        plsc.store_scatter(xidx, [rowv, colv], (fvec + F) * B + bvec)

    def yx_planes(src_hbm, idxref):
        def chunk(t, p):
            @pl.when(t >= 2)
            def _():
                pltpu.make_async_copy(
                    ystage.at[p], out_hbm.at[idxref.at[0]], ysem.at[p]).wait()

            pltpu.async_copy(src_hbm.at[pl.ds(b0 * F + t * YR, YR)],
                             ystage.at[p], ssem.at[p])
            pltpu.make_async_copy(
                src_hbm.at[pl.ds(b0 * F + t * YR, YR)], ystage.at[p],
                ssem.at[p]).wait()
            pltpu.async_copy(ystage.at[p], out_hbm.at[idxref.at[t]],
                             ysem.at[p])

        def two(h, carry):
            chunk(2 * h, 0)
            chunk(2 * h + 1, 1)
            return carry

        lax.fori_loop(0, NYT // 2, two, 0, unroll=False)
        for p in range(2):
            pltpu.make_async_copy(
                ystage.at[p], out_hbm.at[idxref.at[0]], ysem.at[p]).wait()

    yx_planes(y_hbm, yidx)
    yx_planes(x_hbm, xidx)

    pltpu.sync_copy(t_hbm.at[:, pl.ds(b0, BPW)], scalbuf.at[pl.ds(0, F)])
    pltpu.sync_copy(az_hbm.at[pl.ds(b0, BPW)], scalbuf.at[F])
    pltpu.sync_copy(el_hbm.at[pl.ds(b0, BPW)], scalbuf.at[F + 1])

    BR = 4
    NBLK = BPW // BR

    def bblock(i, p):
        pi = lax.div(i, NBLK)
        blk = lax.rem(i, NBLK)

        @pl.when(i >= 2)
        def _():
            pltpu.make_async_copy(
                planebuf.at[p], out_hbm.at[pl.ds(0, BR), :],
                bsem.at[p]).wait()

        def fill_row(r, carry):
            v = plsc.load_gather(
                scalbuf, [jnp.full((16,), pi, jnp.int32),
                          jnp.full((16,), blk * BR + r, jnp.int32)])
            for k in range(N_PV // 16):
                planebuf[p, r, pl.ds(k * 16, 16)] = v
            return carry

        lax.fori_loop(0, BR, fill_row, 0, unroll=False)
        pltpu.async_copy(
            planebuf.at[p],
            out_hbm.at[pl.ds((2 * F + pi) * B + b0 + blk * BR, BR), :],
            bsem.at[p])

    def btwo(h, carry):
        bblock(2 * h, 0)
        bblock(2 * h + 1, 1)
        return carry

    lax.fori_loop(0, 10 * NBLK // 2, btwo, 0, unroll=False)

    def fire_idx(c, p):
        pltpu.async_copy(idx_hbm.at[pl.ds(c * CH, CH)], idxbuf.at[p],
                         isem.at[p])

    def wait_idx(c, p):
        pltpu.make_async_copy(idx_hbm.at[pl.ds(c * CH, CH)], idxbuf.at[p],
                              isem.at[p]).wait()

    def fire_out(plane, c, p):
        pltpu.async_copy(gatbuf.at[p],
                         out_hbm.at[pl.ds(plane * B + c * (CH // N_PV),
                                          CH // N_PV), :],
                         osem.at[p])

    def wait_out(p):
        pltpu.make_async_copy(gatbuf.at[p],
                              out_hbm.at[pl.ds(0, CH // N_PV), :],
                              osem.at[p]).wait()

    def gather_chunk(p):
        def rowfn(r, carry):
            base = r * N_PV
            for j in range(N_PV // 16):
                iv = idxbuf[p, pl.ds(base + j * 16, 16)]
                vals = plsc.load_gather(rowbuf, [iv])
                gatbuf[p, r, pl.ds(j * 16, 16)] = vals
            return carry

        lax.fori_loop(0, CH // N_PV, rowfn, 0, unroll=False)

    def emb_plane(e_row, plane):
        pltpu.sync_copy(table_hbm.at[e_row, pl.ds(NUM_GSPS, VSLICE)],
                        rowbuf.at[pl.ds(0, VSLICE)])
        fire_idx(0, 0)

        def pair(g, carry):
            c0 = 2 * g
            fire_idx(c0 + 1, 1)
            wait_idx(c0, 0)

            @pl.when(c0 >= 2)
            def _():
                wait_out(0)

            gather_chunk(0)
            fire_out(plane, c0, 0)

            @pl.when(c0 + 2 < NCH)
            def _():
                fire_idx(c0 + 2, 0)

            wait_idx(c0 + 1, 1)

            @pl.when(c0 + 1 >= 2)
            def _():
                wait_out(1)

            gather_chunk(1)
            fire_out(plane, c0 + 1, 1)
            return carry

        lax.fori_loop(0, NCH // 2, pair, 0, unroll=False)
        wait_out(0)
        wait_out(1)

    e0 = 2 * wid
    emb_plane(e0, 3 * F + 2 + e0)
    emb_plane(e0 + 1, 3 * F + 2 + e0 + 1)

    for p in range(2):
        pltpu.make_async_copy(planebuf.at[p], out_hbm.at[pl.ds(0, BR), :],
                              bsem.at[p]).wait()


@functools.partial(jax.jit, static_argnames=("interpret",))
def _pv_query(y_t, x_t, idx_flat, t_t, az, el, table_t, interpret=False):
    mesh = plsc.VectorSubcoreMesh(core_axis_name="c", subcore_axis_name="s",
                                  num_cores=NC, num_subcores=NS)
    fn = pl.kernel(
        _sc_body,
        out_type=jax.ShapeDtypeStruct((OUT_D * B, N_PV), jnp.float32),
        mesh=mesh,
        scratch_types=[
            pltpu.VMEM((VSLICE,), jnp.float32),
            pltpu.VMEM((2, CH), jnp.int32),
            pltpu.VMEM((2, CH // N_PV, N_PV), jnp.float32),
            pltpu.VMEM((2, 4, N_PV), jnp.float32),
            pltpu.VMEM((2, YR, N_PV), jnp.float32),
            pltpu.VMEM((F + 2, BPW), jnp.float32),
            pltpu.VMEM((NYT, YR), jnp.int32),
            pltpu.VMEM((NYT, YR), jnp.int32),
            pltpu.SemaphoreType.DMA((2,)),
            pltpu.SemaphoreType.DMA((2,)),
            pltpu.SemaphoreType.DMA((2,)),
            pltpu.SemaphoreType.DMA((2,)),
            pltpu.SemaphoreType.DMA((2,)),
        ],
        compiler_params=pltpu.CompilerParams(use_tc_tiling_on_sc=False,
                                             needs_layout_passes=False),
        interpret=interpret,
    )
    return fn(y_t, x_t, idx_flat, t_t, az, el, table_t)


def kernel(pv_y_osgb_fourier, pv_x_osgb_fourier, pv_system_row_number,
           pv_x_osgb, pv_time_utc_fourier, solar_azimuth, solar_elevation,
           embedding_table):
    del pv_x_osgb
    y_t = jnp.transpose(pv_y_osgb_fourier, (0, 2, 1)).reshape(B * F, N_PV)
    x_t = jnp.transpose(pv_x_osgb_fourier, (0, 2, 1)).reshape(B * F, N_PV)
    idx_flat = pv_system_row_number.astype(jnp.int32).reshape(ROWS)
    t_t = jnp.transpose(pv_time_utc_fourier)
    table_t = jnp.transpose(embedding_table)
    out = _pv_query(y_t, x_t, idx_flat, t_t, solar_azimuth, solar_elevation,
                    table_t)
    return jnp.transpose(out.reshape(OUT_D, B, N_PV), (1, 2, 0))
